# baseline (speedup 1.0000x reference)
import functools
import math

import jax
import jax.numpy as jnp
from jax import lax
from jax.experimental import pallas as pl
from jax.experimental.pallas import tpu as pltpu

_DECAY = 0.99
_LR = 0.01
_CHUNK = 512
_UNROLL = 2


def _fwd_kernel(x_ref, wkvq_ref, bkvq_ref, wo_ref, bo_ref, y_ref,
                state_out_ref, state_sc, mask_sc, dvec_sc, wvec_sc,
                *, C, M, n_steps, unroll, ln_decay):
    c = pl.program_id(1)

    @pl.when(c == 0)
    def _():
        ii = lax.broadcasted_iota(jnp.int32, (C, C), 0)
        jj = lax.broadcasted_iota(jnp.int32, (C, C), 1)
        e = (ii - 1 - jj).astype(jnp.float32)
        mask_sc[...] = jnp.where(jj < ii, jnp.exp(e * ln_decay), 0.0)
        i_c = lax.broadcasted_iota(jnp.int32, (C, M), 0).astype(jnp.float32)
        dvec_sc[...] = jnp.exp(i_c * ln_decay)
        wvec_sc[...] = jnp.exp((C - 1.0 - i_c) * ln_decay)
        state_sc[...] = jnp.zeros_like(state_sc)

    c11 = (((1,), (1,)), ((), ()))
    for h in range(unroll):
        xc = x_ref[0, h * C:(h + 1) * C, :].astype(jnp.bfloat16)
        kvq = lax.dot_general(xc, wkvq_ref[...], c11,
                              preferred_element_type=jnp.float32) + bkvq_ref[...]
        kb = kvq[:, :M].astype(jnp.bfloat16)
        v = kvq[:, M:2 * M]
        vb = v.astype(jnp.bfloat16)
        qb = kvq[:, 2 * M:].astype(jnp.bfloat16)

        inter = lax.dot_general(qb, state_sc[...].astype(jnp.bfloat16), c11,
                                preferred_element_type=jnp.float32) * dvec_sc[...]

        a = lax.dot_general(qb, kb, c11,
                            preferred_element_type=jnp.float32) * mask_sc[...]
        intra = lax.dot_general(a.astype(jnp.bfloat16), vb,
                                (((1,), (0,)), ((), ())),
                                preferred_element_type=jnp.float32)

        outs = inter + _LR * intra
        y_ref[0, h * C:(h + 1) * C, :] = lax.dot_general(
            outs.astype(jnp.bfloat16), wo_ref[...], c11,
            preferred_element_type=jnp.float32) + bo_ref[...]

        supd = lax.dot_general((v * wvec_sc[...]).astype(jnp.bfloat16), kb,
                               (((0,), (0,)), ((), ())),
                               preferred_element_type=jnp.float32)
        state_sc[...] = (_DECAY ** C) * state_sc[...] + _LR * supd

    @pl.when(c == n_steps - 1)
    def _():
        state_out_ref[0] = state_sc[...]


def kernel(x, Wk, bk, Wv, bv, Wq, bq, Wo, bo):
    B, S, D = x.shape
    M = Wk.shape[0]
    C = _CHUNK
    R = _UNROLL
    assert S % (C * R) == 0
    n_steps = S // (C * R)
    wkvq = jnp.concatenate([Wk, Wv, Wq], axis=0).astype(jnp.bfloat16)
    bkvq = jnp.concatenate([bk, bv, bq], axis=0).reshape(1, 3 * M)
    body = functools.partial(_fwd_kernel, C=C, M=M, n_steps=n_steps,
                             unroll=R, ln_decay=math.log(_DECAY))
    y, state = pl.pallas_call(
        body,
        grid=(B, n_steps),
        in_specs=[
            pl.BlockSpec((1, C * R, D), lambda b, c: (b, c, 0)),
            pl.BlockSpec((3 * M, D), lambda b, c: (0, 0)),
            pl.BlockSpec((1, 3 * M), lambda b, c: (0, 0)),
            pl.BlockSpec((D, M), lambda b, c: (0, 0)),
            pl.BlockSpec((1, D), lambda b, c: (0, 0)),
        ],
        out_specs=[
            pl.BlockSpec((1, C * R, D), lambda b, c: (b, c, 0)),
            pl.BlockSpec((1, M, M), lambda b, c: (b, 0, 0)),
        ],
        out_shape=[
            jax.ShapeDtypeStruct((B, S, D), jnp.float32),
            jax.ShapeDtypeStruct((B, M, M), jnp.float32),
        ],
        scratch_shapes=[
            pltpu.VMEM((M, M), jnp.float32),
            pltpu.VMEM((C, C), jnp.float32),
            pltpu.VMEM((C, M), jnp.float32),
            pltpu.VMEM((C, M), jnp.float32),
        ],
        compiler_params=pltpu.CompilerParams(
            dimension_semantics=("parallel", "arbitrary"),
        ),
    )(x, wkvq, bkvq, Wo.astype(jnp.bfloat16), bo.reshape(1, D))
    return (y, state)


# speedup vs baseline: 1.0463x; 1.0463x over previous
import functools
import math

import jax
import jax.numpy as jnp
from jax import lax
from jax.experimental import pallas as pl
from jax.experimental.pallas import tpu as pltpu

_DECAY = 0.99
_LR = 0.01
_CHUNK = 512
_UNROLL = 4


def _fwd_kernel(x_ref, wkvq_ref, bkvq_ref, wo_ref, bo_ref, y_ref,
                state_out_ref, state_sc, mask_sc, dvec_sc, wvec_sc,
                *, C, M, n_steps, unroll, ln_decay):
    c = pl.program_id(1)

    @pl.when(c == 0)
    def _():
        ii = lax.broadcasted_iota(jnp.int32, (C, C), 0)
        jj = lax.broadcasted_iota(jnp.int32, (C, C), 1)
        e = (ii - 1 - jj).astype(jnp.float32)
        mask_sc[...] = jnp.where(jj < ii, jnp.exp(e * ln_decay), 0.0)
        i_c = lax.broadcasted_iota(jnp.int32, (C, M), 0).astype(jnp.float32)
        dvec_sc[...] = jnp.exp(i_c * ln_decay)
        wvec_sc[...] = jnp.exp((C - 1.0 - i_c) * ln_decay)
        state_sc[...] = jnp.zeros_like(state_sc)

    c11 = (((1,), (1,)), ((), ()))
    for h in range(unroll):
        xc = x_ref[0, h * C:(h + 1) * C, :].astype(jnp.bfloat16)
        kvq = lax.dot_general(xc, wkvq_ref[...], c11,
                              preferred_element_type=jnp.float32) + bkvq_ref[...]
        kb = kvq[:, :M].astype(jnp.bfloat16)
        v = kvq[:, M:2 * M]
        vb = v.astype(jnp.bfloat16)
        qb = kvq[:, 2 * M:].astype(jnp.bfloat16)

        inter = lax.dot_general(qb, state_sc[...].astype(jnp.bfloat16), c11,
                                preferred_element_type=jnp.float32) * dvec_sc[...]

        a = lax.dot_general(qb, kb, c11,
                            preferred_element_type=jnp.float32) * mask_sc[...]
        intra = lax.dot_general(a.astype(jnp.bfloat16), vb,
                                (((1,), (0,)), ((), ())),
                                preferred_element_type=jnp.float32)

        outs = inter + _LR * intra
        y_ref[0, h * C:(h + 1) * C, :] = lax.dot_general(
            outs.astype(jnp.bfloat16), wo_ref[...], c11,
            preferred_element_type=jnp.float32) + bo_ref[...]

        supd = lax.dot_general((v * wvec_sc[...]).astype(jnp.bfloat16), kb,
                               (((0,), (0,)), ((), ())),
                               preferred_element_type=jnp.float32)
        state_sc[...] = (_DECAY ** C) * state_sc[...] + _LR * supd

    @pl.when(c == n_steps - 1)
    def _():
        state_out_ref[0] = state_sc[...]


def kernel(x, Wk, bk, Wv, bv, Wq, bq, Wo, bo):
    B, S, D = x.shape
    M = Wk.shape[0]
    C = _CHUNK
    R = _UNROLL
    assert S % (C * R) == 0
    n_steps = S // (C * R)
    wkvq = jnp.concatenate([Wk, Wv, Wq], axis=0).astype(jnp.bfloat16)
    bkvq = jnp.concatenate([bk, bv, bq], axis=0).reshape(1, 3 * M)
    body = functools.partial(_fwd_kernel, C=C, M=M, n_steps=n_steps,
                             unroll=R, ln_decay=math.log(_DECAY))
    y, state = pl.pallas_call(
        body,
        grid=(B, n_steps),
        in_specs=[
            pl.BlockSpec((1, C * R, D), lambda b, c: (b, c, 0)),
            pl.BlockSpec((3 * M, D), lambda b, c: (0, 0)),
            pl.BlockSpec((1, 3 * M), lambda b, c: (0, 0)),
            pl.BlockSpec((D, M), lambda b, c: (0, 0)),
            pl.BlockSpec((1, D), lambda b, c: (0, 0)),
        ],
        out_specs=[
            pl.BlockSpec((1, C * R, D), lambda b, c: (b, c, 0)),
            pl.BlockSpec((1, M, M), lambda b, c: (b, 0, 0)),
        ],
        out_shape=[
            jax.ShapeDtypeStruct((B, S, D), jnp.float32),
            jax.ShapeDtypeStruct((B, M, M), jnp.float32),
        ],
        scratch_shapes=[
            pltpu.VMEM((M, M), jnp.float32),
            pltpu.VMEM((C, C), jnp.float32),
            pltpu.VMEM((C, M), jnp.float32),
            pltpu.VMEM((C, M), jnp.float32),
        ],
        compiler_params=pltpu.CompilerParams(
            dimension_semantics=("parallel", "arbitrary"),
        ),
    )(x, wkvq, bkvq, Wo.astype(jnp.bfloat16), bo.reshape(1, D))
    return (y, state)
